# baseline (speedup 1.0000x reference)
import functools

import jax
import jax.numpy as jnp
from jax import lax
from jax.experimental import pallas as pl
from jax.experimental.pallas import tpu as pltpu
from jax.experimental.pallas import tpu_sc as plsc

NC = 2
NS = 16
NW = NC * NS
CB = 128

_MAGIC = 0x5F3759DF


def _quake_rsqrt(v):
    ib = plsc.bitcast(v, jnp.int32)
    y = plsc.bitcast(jnp.full((16,), _MAGIC, jnp.int32)
                     - lax.shift_right_logical(ib, 1), jnp.float32)
    for _ in range(3):
        y = y * (1.5 - 0.5 * v * y * y)
    return y


def _make_deg(np_rows, e):
    et = e // NW
    kf = et // CB
    tail = et - kf * CB
    rpt = np_rows // NS
    mesh = plsc.VectorSubcoreMesh(core_axis_name="c", subcore_axis_name="s")

    @functools.partial(
        pl.kernel,
        out_type=jax.ShapeDtypeStruct((NC, np_rows, 16), jnp.float32),
        mesh=mesh,
        compiler_params=pltpu.CompilerParams(use_tc_tiling_on_sc=False),
        scratch_types=[
            pltpu.VMEM((et,), jnp.int32),
            pltpu.VMEM((CB, 16), jnp.float32),
            pltpu.VMEM_SHARED((np_rows, 16), jnp.float32),
            pltpu.SemaphoreType.DMA,
        ],
    )
    def deg_kernel(ei_hbm, zeros_hbm, ones_hbm, out_hbm, dst_v, ones_v, acc,
                   sem):
        c = lax.axis_index("c")
        s = lax.axis_index("s")
        wid = c * NS + s
        pltpu.sync_copy(zeros_hbm.at[pl.ds(s * rpt, rpt)],
                        acc.at[pl.ds(s * rpt, rpt)])
        pltpu.sync_copy(ei_hbm.at[1, pl.ds(wid * et, et)], dst_v)
        pltpu.sync_copy(ones_hbm, ones_v)
        plsc.subcore_barrier()

        nbd = 4

        def body(j, carry):
            pltpu.async_copy(ones_v, acc.at[dst_v.at[pl.ds(j * CB, CB)]],
                             sem, add=True)

            @pl.when(j >= nbd)
            def _():
                pltpu.make_async_copy(ones_v,
                                      acc.at[dst_v.at[pl.ds(0, CB)]],
                                      sem).wait()

            return carry

        lax.fori_loop(0, kf, body, 0)
        for _ in range(min(nbd, kf)):
            pltpu.make_async_copy(ones_v, acc.at[dst_v.at[pl.ds(0, CB)]],
                                  sem).wait()
        if tail:
            pltpu.sync_copy(ones_v.at[pl.ds(0, tail)],
                            acc.at[dst_v.at[pl.ds(kf * CB, tail)]], add=True)
        plsc.subcore_barrier()
        pltpu.sync_copy(acc.at[pl.ds(s * rpt, rpt)],
                        out_hbm.at[c, pl.ds(s * rpt, rpt)])

    return deg_kernel


def _make_prescale(n, np_rows, d):
    dh = d // 2
    rpt = np_rows // NS
    last = NS - 1
    last_cnt = n - last * rpt
    mesh = plsc.VectorSubcoreMesh(core_axis_name="c", subcore_axis_name="s")
    sub = rpt // 2

    @functools.partial(
        pl.kernel,
        out_type=[
            jax.ShapeDtypeStruct((NC, n, dh), jnp.float32),
            jax.ShapeDtypeStruct((n, 16), jnp.float32),
        ],
        mesh=mesh,
        compiler_params=pltpu.CompilerParams(use_tc_tiling_on_sc=False,
                                             needs_layout_passes=False),
        scratch_types=[
            pltpu.VMEM((sub, d), jnp.float32),
            pltpu.VMEM((sub, dh), jnp.float32),
            pltpu.VMEM((sub, 16), jnp.float32),
            pltpu.VMEM((sub, 16), jnp.float32),
        ],
    )
    def prescale_kernel(degp_hbm, x_hbm, g1_hbm, dinv_hbm,
                        x_v, g1_v, d0_v, d1_v):
        c = lax.axis_index("c")
        s = lax.axis_index("s")

        def run(cnt, coff):
            for off in range(0, cnt, sub):
                c2 = min(sub, cnt - off)
                r0 = s * rpt + off
                pltpu.sync_copy(x_hbm.at[pl.ds(r0, c2)],
                                x_v.at[pl.ds(0, c2)])
                pltpu.sync_copy(degp_hbm.at[0, pl.ds(r0, c2)],
                                d0_v.at[pl.ds(0, c2)])
                pltpu.sync_copy(degp_hbm.at[1, pl.ds(r0, c2)],
                                d1_v.at[pl.ds(0, c2)])

                def rowfn(i, carry):
                    y = _quake_rsqrt(d0_v[i, :] + d1_v[i, :] + 1.0)
                    d0_v[i, :] = y
                    for kk in range(dh // 16):
                        sl = pl.ds(kk * 16, 16)
                        g1_v[i, sl] = x_v[i, pl.ds(coff + kk * 16, 16)] * y
                    return carry

                lax.fori_loop(0, c2, rowfn, 0)
                pltpu.sync_copy(g1_v.at[pl.ds(0, c2)],
                                g1_hbm.at[c, pl.ds(r0, c2)])

                @pl.when(c == 0)
                def _():
                    pltpu.sync_copy(d0_v.at[pl.ds(0, c2)],
                                    dinv_hbm.at[pl.ds(r0, c2)])

        for cc in range(NC):
            @pl.when((c == cc) & (s < last))
            def _(cc=cc):
                run(rpt, cc * dh)

            @pl.when((c == cc) & (s == last))
            def _(cc=cc):
                run(last_cnt, cc * dh)

    return prescale_kernel


def _ring_propagate(gh, src_v, dst_v, rows_v, acc, gsem, ssem, kf, tail, nb):
    for b in range(nb):
        pltpu.async_copy(gh.at[src_v.at[pl.ds(b * CB, CB)]], rows_v.at[b],
                         gsem)

    def body(j, carry):
        bj = lax.rem(j, nb)
        pltpu.make_async_copy(gh.at[src_v.at[pl.ds(bj * CB, CB)]],
                              rows_v.at[bj], gsem).wait()
        pltpu.async_copy(rows_v.at[bj], acc.at[dst_v.at[pl.ds(j * CB, CB)]],
                         ssem, add=True)
        nxt = j + (nb - 1)

        @pl.when((j >= 1) & (nxt < kf))
        def _():
            bp = lax.rem(nxt, nb)
            pltpu.make_async_copy(rows_v.at[bp],
                                  acc.at[dst_v.at[pl.ds(0, CB)]], ssem).wait()
            pltpu.async_copy(gh.at[src_v.at[pl.ds(nxt * CB, CB)]],
                             rows_v.at[bp], gsem)

        return carry

    lax.fori_loop(0, kf, body, 0)
    for _ in range(nb):
        pltpu.make_async_copy(rows_v.at[0], acc.at[dst_v.at[pl.ds(0, CB)]],
                              ssem).wait()
    if tail:
        t0 = kf * CB
        pltpu.async_copy(gh.at[src_v.at[pl.ds(t0, tail)]],
                         rows_v.at[0, pl.ds(0, tail)], gsem).wait()
        pltpu.sync_copy(rows_v.at[0, pl.ds(0, tail)],
                        acc.at[dst_v.at[pl.ds(t0, tail)]], add=True)


def _init_acc_from_g(acc, g_hbm, zeros_hbm, c, s, n, rpt, last, last_cnt):
    r0 = s * rpt

    @pl.when(s < last)
    def _():
        pltpu.sync_copy(g_hbm.at[c, pl.ds(r0, rpt)], acc.at[pl.ds(r0, rpt)])

    @pl.when(s == last)
    def _():
        pltpu.sync_copy(g_hbm.at[c, pl.ds(r0, last_cnt)],
                        acc.at[pl.ds(r0, last_cnt)])
        pltpu.sync_copy(zeros_hbm.at[pl.ds(0, rpt - last_cnt)],
                        acc.at[pl.ds(r0 + last_cnt, rpt - last_cnt)])


def _make_prop(n, np_rows, dh, e, nb):
    et = e // NS
    kf = et // CB
    tail = et - kf * CB
    rpt = np_rows // NS
    last = NS - 1
    last_cnt = n - last * rpt
    mesh = plsc.VectorSubcoreMesh(core_axis_name="c", subcore_axis_name="s")

    @functools.partial(
        pl.kernel,
        out_type=jax.ShapeDtypeStruct((NC, np_rows, dh), jnp.float32),
        mesh=mesh,
        compiler_params=pltpu.CompilerParams(use_tc_tiling_on_sc=False),
        scratch_types=[
            pltpu.VMEM((et,), jnp.int32),
            pltpu.VMEM((et,), jnp.int32),
            pltpu.VMEM((nb, CB, dh), jnp.float32),
            pltpu.VMEM_SHARED((np_rows, dh), jnp.float32),
            pltpu.SemaphoreType.DMA,
            pltpu.SemaphoreType.DMA,
        ],
    )
    def prop_kernel(ei_hbm, g_hbm, zeros_hbm, out_hbm,
                    src_v, dst_v, rows_v, acc, gsem, ssem):
        c = lax.axis_index("c")
        s = lax.axis_index("s")
        _init_acc_from_g(acc, g_hbm, zeros_hbm, c, s, n, rpt, last, last_cnt)
        pltpu.sync_copy(ei_hbm.at[0, pl.ds(s * et, et)], src_v)
        pltpu.sync_copy(ei_hbm.at[1, pl.ds(s * et, et)], dst_v)
        plsc.subcore_barrier()
        _ring_propagate(g_hbm.at[c], src_v, dst_v, rows_v, acc,
                        gsem, ssem, kf, tail, nb)
        plsc.subcore_barrier()
        pltpu.sync_copy(acc.at[pl.ds(s * rpt, rpt)],
                        out_hbm.at[c, pl.ds(s * rpt, rpt)])

    return prop_kernel


def _make_prop_final(n, np_rows, dh, e, nb):
    et = e // NS
    kf = et // CB
    tail = et - kf * CB
    rpt = np_rows // NS
    last = NS - 1
    last_cnt = n - last * rpt
    mesh = plsc.VectorSubcoreMesh(core_axis_name="c", subcore_axis_name="s")

    @functools.partial(
        pl.kernel,
        out_type=jax.ShapeDtypeStruct((NC, n, dh), jnp.float32),
        mesh=mesh,
        compiler_params=pltpu.CompilerParams(use_tc_tiling_on_sc=False),
        scratch_types=[
            pltpu.VMEM((et,), jnp.int32),
            pltpu.VMEM((et,), jnp.int32),
            pltpu.VMEM((nb, CB, dh), jnp.float32),
            pltpu.VMEM((rpt, dh), jnp.float32),
            pltpu.VMEM((rpt, 16), jnp.float32),
            pltpu.VMEM((dh,), jnp.float32),
            pltpu.VMEM_SHARED((np_rows, dh), jnp.float32),
            pltpu.SemaphoreType.DMA,
            pltpu.SemaphoreType.DMA,
        ],
    )
    def prop_kernel(ei_hbm, g_hbm, zeros_hbm, dinv_hbm, bias_hbm,
                    out_hbm, src_v, dst_v, rows_v, res_v, dinv_v,
                    bias_v, acc, gsem, ssem):
        c = lax.axis_index("c")
        s = lax.axis_index("s")
        _init_acc_from_g(acc, g_hbm, zeros_hbm, c, s, n, rpt, last, last_cnt)
        pltpu.sync_copy(ei_hbm.at[0, pl.ds(s * et, et)], src_v)
        pltpu.sync_copy(ei_hbm.at[1, pl.ds(s * et, et)], dst_v)
        pltpu.sync_copy(bias_hbm.at[c], bias_v)
        plsc.subcore_barrier()
        _ring_propagate(g_hbm.at[c], src_v, dst_v, rows_v, acc,
                        gsem, ssem, kf, tail, nb)
        plsc.subcore_barrier()

        r0 = s * rpt
        pltpu.sync_copy(acc.at[pl.ds(r0, rpt)], res_v)
        bias = bias_v[:]

        def fin(i, carry):
            for kk in range(dh // 16):
                sl = pl.ds(kk * 16, 16)
                res_v[i, sl] = res_v[i, sl] * dinv_v[i, pl.ds(0, 16)] + bias
            return carry

        @pl.when(s < last)
        def _():
            pltpu.sync_copy(dinv_hbm.at[pl.ds(r0, rpt)], dinv_v)
            lax.fori_loop(0, rpt, fin, 0)
            pltpu.sync_copy(res_v, out_hbm.at[c, pl.ds(r0, rpt)])

        @pl.when(s == last)
        def _():
            pltpu.sync_copy(dinv_hbm.at[pl.ds(r0, last_cnt)],
                            dinv_v.at[pl.ds(0, last_cnt)])
            lax.fori_loop(0, last_cnt, fin, 0)
            pltpu.sync_copy(res_v.at[pl.ds(0, last_cnt)],
                            out_hbm.at[c, pl.ds(r0, last_cnt)])

    return prop_kernel


def _mlp_body(dinv16_ref, s1_ref, w1_ref, b1_ref, w2_ref, g2_ref):
    dinv = dinv16_ref[:, 0:1]
    a1 = jnp.concatenate([s1_ref[0], s1_ref[1]], axis=1) * dinv
    h = jnp.dot(a1, w1_ref[...], preferred_element_type=jnp.float32)
    h = jnp.maximum(h + b1_ref[...], 0.0)
    t = jnp.dot(h, w2_ref[...], preferred_element_type=jnp.float32)
    g2 = t * dinv
    ch = t.shape[1] // 2
    g2_ref[0] = g2[:, :ch]
    g2_ref[1] = g2[:, ch:]


def kernel(x, edge_index, W1, b1, W2, b2):
    n, d = x.shape
    h_dim = W1.shape[1]
    c_dim = W2.shape[1]
    e = edge_index.shape[1]
    dh = d // 2

    np_rows = -(-n // (NS * 8)) * NS * 8

    z16 = jnp.zeros((np_rows, 16), jnp.float32)
    zd = jnp.zeros((np_rows // NS, dh), jnp.float32)
    zc = jnp.zeros((np_rows // NS, c_dim // 2), jnp.float32)
    ones = jnp.ones((CB, 16), jnp.float32)

    degp = _make_deg(np_rows, e)(edge_index, z16, ones)

    g1, dinv16 = _make_prescale(n, np_rows, d)(degp, x)

    s1 = _make_prop(n, np_rows, dh, e, 5)(edge_index, g1, zd)

    bn = 1000
    grid = (n // bn,)
    g2 = pl.pallas_call(
        _mlp_body,
        grid=grid,
        in_specs=[
            pl.BlockSpec((bn, 16), lambda i: (i, 0)),
            pl.BlockSpec((2, bn, dh), lambda i: (0, i, 0)),
            pl.BlockSpec((d, h_dim), lambda i: (0, 0)),
            pl.BlockSpec((1, h_dim), lambda i: (0, 0)),
            pl.BlockSpec((h_dim, c_dim), lambda i: (0, 0)),
        ],
        out_specs=pl.BlockSpec((2, bn, c_dim // 2), lambda i: (0, i, 0)),
        out_shape=jax.ShapeDtypeStruct((2, n, c_dim // 2), jnp.float32),
    )(dinv16, s1, W1, b1.reshape(1, h_dim), W2)

    b2h = b2.reshape(NC, c_dim // 2)
    outh = _make_prop_final(n, np_rows, c_dim // 2, e, 8)(
        edge_index, g2, zc, dinv16, b2h)
    return jnp.concatenate([outh[0], outh[1]], axis=1)


# speedup vs baseline: 1.0441x; 1.0441x over previous
import functools

import jax
import jax.numpy as jnp
from jax import lax
from jax.experimental import pallas as pl
from jax.experimental.pallas import tpu as pltpu
from jax.experimental.pallas import tpu_sc as plsc

NC = 2
NS = 16
NW = NC * NS
CB = 128

def _make_deg(np_rows, e):
    et = e // NW
    kf = et // CB
    tail = et - kf * CB
    rpt = np_rows // NS
    mesh = plsc.VectorSubcoreMesh(core_axis_name="c", subcore_axis_name="s")

    @functools.partial(
        pl.kernel,
        out_type=jax.ShapeDtypeStruct((NC, np_rows, 16), jnp.float32),
        mesh=mesh,
        compiler_params=pltpu.CompilerParams(use_tc_tiling_on_sc=False),
        scratch_types=[
            pltpu.VMEM((et,), jnp.int32),
            pltpu.VMEM((CB, 16), jnp.float32),
            pltpu.VMEM_SHARED((np_rows, 16), jnp.float32),
            pltpu.SemaphoreType.DMA,
        ],
    )
    def deg_kernel(ei_hbm, zeros_hbm, ones_hbm, out_hbm, dst_v, ones_v, acc,
                   sem):
        c = lax.axis_index("c")
        s = lax.axis_index("s")
        wid = c * NS + s
        pltpu.sync_copy(zeros_hbm.at[pl.ds(s * rpt, rpt)],
                        acc.at[pl.ds(s * rpt, rpt)])
        pltpu.sync_copy(ei_hbm.at[1, pl.ds(wid * et, et)], dst_v)
        pltpu.sync_copy(ones_hbm, ones_v)
        plsc.subcore_barrier()

        nbd = 4

        def body(j, carry):
            pltpu.async_copy(ones_v, acc.at[dst_v.at[pl.ds(j * CB, CB)]],
                             sem, add=True)

            @pl.when(j >= nbd)
            def _():
                pltpu.make_async_copy(ones_v,
                                      acc.at[dst_v.at[pl.ds(0, CB)]],
                                      sem).wait()

            return carry

        lax.fori_loop(0, kf, body, 0)
        for _ in range(min(nbd, kf)):
            pltpu.make_async_copy(ones_v, acc.at[dst_v.at[pl.ds(0, CB)]],
                                  sem).wait()
        if tail:
            pltpu.sync_copy(ones_v.at[pl.ds(0, tail)],
                            acc.at[dst_v.at[pl.ds(kf * CB, tail)]], add=True)
        plsc.subcore_barrier()
        pltpu.sync_copy(acc.at[pl.ds(s * rpt, rpt)],
                        out_hbm.at[c, pl.ds(s * rpt, rpt)])

    return deg_kernel


def _prescale_body(degp_ref, x_ref, g1_ref, dinv_ref):
    deg = degp_ref[0, :, 0:1] + degp_ref[1, :, 0:1] + 1.0
    dinv = lax.rsqrt(deg)
    d = x_ref.shape[1]
    g1_ref[0] = x_ref[:, : d // 2] * dinv
    g1_ref[1] = x_ref[:, d // 2:] * dinv
    dinv_ref[...] = jnp.broadcast_to(dinv, dinv_ref.shape)


def _ring_propagate(gh, src_v, dst_v, rows_v, acc, gsem, ssem, kf, tail, nb):
    for b in range(nb):
        pltpu.async_copy(gh.at[src_v.at[pl.ds(b * CB, CB)]], rows_v.at[b],
                         gsem)

    def body(j, carry):
        bj = lax.rem(j, nb)
        pltpu.make_async_copy(gh.at[src_v.at[pl.ds(bj * CB, CB)]],
                              rows_v.at[bj], gsem).wait()
        pltpu.async_copy(rows_v.at[bj], acc.at[dst_v.at[pl.ds(j * CB, CB)]],
                         ssem, add=True)
        nxt = j + (nb - 1)

        @pl.when((j >= 1) & (nxt < kf))
        def _():
            bp = lax.rem(nxt, nb)
            pltpu.make_async_copy(rows_v.at[bp],
                                  acc.at[dst_v.at[pl.ds(0, CB)]], ssem).wait()
            pltpu.async_copy(gh.at[src_v.at[pl.ds(nxt * CB, CB)]],
                             rows_v.at[bp], gsem)

        return carry

    lax.fori_loop(0, kf, body, 0)
    for _ in range(nb):
        pltpu.make_async_copy(rows_v.at[0], acc.at[dst_v.at[pl.ds(0, CB)]],
                              ssem).wait()
    if tail:
        t0 = kf * CB
        pltpu.async_copy(gh.at[src_v.at[pl.ds(t0, tail)]],
                         rows_v.at[0, pl.ds(0, tail)], gsem).wait()
        pltpu.sync_copy(rows_v.at[0, pl.ds(0, tail)],
                        acc.at[dst_v.at[pl.ds(t0, tail)]], add=True)


def _init_acc_from_g(acc, g_hbm, zeros_hbm, c, s, n, rpt, last, last_cnt):
    r0 = s * rpt

    @pl.when(s < last)
    def _():
        pltpu.sync_copy(g_hbm.at[c, pl.ds(r0, rpt)], acc.at[pl.ds(r0, rpt)])

    @pl.when(s == last)
    def _():
        pltpu.sync_copy(g_hbm.at[c, pl.ds(r0, last_cnt)],
                        acc.at[pl.ds(r0, last_cnt)])
        pltpu.sync_copy(zeros_hbm.at[pl.ds(0, rpt - last_cnt)],
                        acc.at[pl.ds(r0 + last_cnt, rpt - last_cnt)])


def _make_prop(n, np_rows, dh, e, nb):
    et = e // NS
    kf = et // CB
    tail = et - kf * CB
    rpt = np_rows // NS
    last = NS - 1
    last_cnt = n - last * rpt
    mesh = plsc.VectorSubcoreMesh(core_axis_name="c", subcore_axis_name="s")

    @functools.partial(
        pl.kernel,
        out_type=jax.ShapeDtypeStruct((NC, np_rows, dh), jnp.float32),
        mesh=mesh,
        compiler_params=pltpu.CompilerParams(use_tc_tiling_on_sc=False),
        scratch_types=[
            pltpu.VMEM((et,), jnp.int32),
            pltpu.VMEM((et,), jnp.int32),
            pltpu.VMEM((nb, CB, dh), jnp.float32),
            pltpu.VMEM_SHARED((np_rows, dh), jnp.float32),
            pltpu.SemaphoreType.DMA,
            pltpu.SemaphoreType.DMA,
        ],
    )
    def prop_kernel(ei_hbm, g_hbm, zeros_hbm, out_hbm,
                    src_v, dst_v, rows_v, acc, gsem, ssem):
        c = lax.axis_index("c")
        s = lax.axis_index("s")
        _init_acc_from_g(acc, g_hbm, zeros_hbm, c, s, n, rpt, last, last_cnt)
        pltpu.sync_copy(ei_hbm.at[0, pl.ds(s * et, et)], src_v)
        pltpu.sync_copy(ei_hbm.at[1, pl.ds(s * et, et)], dst_v)
        plsc.subcore_barrier()
        _ring_propagate(g_hbm.at[c], src_v, dst_v, rows_v, acc,
                        gsem, ssem, kf, tail, nb)
        plsc.subcore_barrier()
        pltpu.sync_copy(acc.at[pl.ds(s * rpt, rpt)],
                        out_hbm.at[c, pl.ds(s * rpt, rpt)])

    return prop_kernel


def _make_prop_final(n, np_rows, dh, e, nb):
    et = e // NS
    kf = et // CB
    tail = et - kf * CB
    rpt = np_rows // NS
    last = NS - 1
    last_cnt = n - last * rpt
    mesh = plsc.VectorSubcoreMesh(core_axis_name="c", subcore_axis_name="s")

    @functools.partial(
        pl.kernel,
        out_type=jax.ShapeDtypeStruct((NC, n, dh), jnp.float32),
        mesh=mesh,
        compiler_params=pltpu.CompilerParams(use_tc_tiling_on_sc=False),
        scratch_types=[
            pltpu.VMEM((et,), jnp.int32),
            pltpu.VMEM((et,), jnp.int32),
            pltpu.VMEM((nb, CB, dh), jnp.float32),
            pltpu.VMEM((rpt, dh), jnp.float32),
            pltpu.VMEM((rpt, 16), jnp.float32),
            pltpu.VMEM((dh,), jnp.float32),
            pltpu.VMEM_SHARED((np_rows, dh), jnp.float32),
            pltpu.SemaphoreType.DMA,
            pltpu.SemaphoreType.DMA,
        ],
    )
    def prop_kernel(ei_hbm, g_hbm, zeros_hbm, dinv_hbm, bias_hbm,
                    out_hbm, src_v, dst_v, rows_v, res_v, dinv_v,
                    bias_v, acc, gsem, ssem):
        c = lax.axis_index("c")
        s = lax.axis_index("s")
        _init_acc_from_g(acc, g_hbm, zeros_hbm, c, s, n, rpt, last, last_cnt)
        pltpu.sync_copy(ei_hbm.at[0, pl.ds(s * et, et)], src_v)
        pltpu.sync_copy(ei_hbm.at[1, pl.ds(s * et, et)], dst_v)
        pltpu.sync_copy(bias_hbm.at[c], bias_v)
        plsc.subcore_barrier()
        _ring_propagate(g_hbm.at[c], src_v, dst_v, rows_v, acc,
                        gsem, ssem, kf, tail, nb)
        plsc.subcore_barrier()

        r0 = s * rpt
        pltpu.sync_copy(acc.at[pl.ds(r0, rpt)], res_v)
        bias = bias_v[:]

        def fin(i, carry):
            for kk in range(dh // 16):
                sl = pl.ds(kk * 16, 16)
                res_v[i, sl] = res_v[i, sl] * dinv_v[i, pl.ds(0, 16)] + bias
            return carry

        @pl.when(s < last)
        def _():
            pltpu.sync_copy(dinv_hbm.at[pl.ds(r0, rpt)], dinv_v)
            lax.fori_loop(0, rpt, fin, 0)
            pltpu.sync_copy(res_v, out_hbm.at[c, pl.ds(r0, rpt)])

        @pl.when(s == last)
        def _():
            pltpu.sync_copy(dinv_hbm.at[pl.ds(r0, last_cnt)],
                            dinv_v.at[pl.ds(0, last_cnt)])
            lax.fori_loop(0, last_cnt, fin, 0)
            pltpu.sync_copy(res_v.at[pl.ds(0, last_cnt)],
                            out_hbm.at[c, pl.ds(r0, last_cnt)])

    return prop_kernel


def _mlp_body(dinv16_ref, s1_ref, w1_ref, b1_ref, w2_ref, g2_ref):
    dinv = dinv16_ref[:, 0:1]
    a1 = jnp.concatenate([s1_ref[0], s1_ref[1]], axis=1) * dinv
    h = jnp.dot(a1, w1_ref[...], preferred_element_type=jnp.float32)
    h = jnp.maximum(h + b1_ref[...], 0.0)
    t = jnp.dot(h, w2_ref[...], preferred_element_type=jnp.float32)
    g2 = t * dinv
    ch = t.shape[1] // 2
    g2_ref[0] = g2[:, :ch]
    g2_ref[1] = g2[:, ch:]


def kernel(x, edge_index, W1, b1, W2, b2):
    n, d = x.shape
    h_dim = W1.shape[1]
    c_dim = W2.shape[1]
    e = edge_index.shape[1]
    dh = d // 2

    np_rows = -(-n // (NS * 8)) * NS * 8

    z16 = jnp.zeros((np_rows, 16), jnp.float32)
    zd = jnp.zeros((np_rows // NS, dh), jnp.float32)
    zc = jnp.zeros((np_rows // NS, c_dim // 2), jnp.float32)
    ones = jnp.ones((CB, 16), jnp.float32)

    degp = _make_deg(np_rows, e)(edge_index, z16, ones)

    bn = 1000
    grid = (n // bn,)
    g1, dinv16 = pl.pallas_call(
        _prescale_body,
        grid=grid,
        in_specs=[
            pl.BlockSpec((2, bn, 16), lambda i: (0, i, 0)),
            pl.BlockSpec((bn, d), lambda i: (i, 0)),
        ],
        out_specs=[
            pl.BlockSpec((2, bn, dh), lambda i: (0, i, 0)),
            pl.BlockSpec((bn, 16), lambda i: (i, 0)),
        ],
        out_shape=[
            jax.ShapeDtypeStruct((2, n, dh), jnp.float32),
            jax.ShapeDtypeStruct((n, 16), jnp.float32),
        ],
    )(degp, x)

    s1 = _make_prop(n, np_rows, dh, e, 6)(edge_index, g1, zd)
    g2 = pl.pallas_call(
        _mlp_body,
        grid=grid,
        in_specs=[
            pl.BlockSpec((bn, 16), lambda i: (i, 0)),
            pl.BlockSpec((2, bn, dh), lambda i: (0, i, 0)),
            pl.BlockSpec((d, h_dim), lambda i: (0, 0)),
            pl.BlockSpec((1, h_dim), lambda i: (0, 0)),
            pl.BlockSpec((h_dim, c_dim), lambda i: (0, 0)),
        ],
        out_specs=pl.BlockSpec((2, bn, c_dim // 2), lambda i: (0, i, 0)),
        out_shape=jax.ShapeDtypeStruct((2, n, c_dim // 2), jnp.float32),
    )(dinv16, s1, W1, b1.reshape(1, h_dim), W2)

    b2h = b2.reshape(NC, c_dim // 2)
    outh = _make_prop_final(n, np_rows, c_dim // 2, e, 8)(
        edge_index, g2, zc, dinv16, b2h)
    return jnp.concatenate([outh[0], outh[1]], axis=1)


# speedup vs baseline: 1.0796x; 1.0340x over previous
import functools

import jax
import jax.numpy as jnp
from jax import lax
from jax.experimental import pallas as pl
from jax.experimental.pallas import tpu as pltpu
from jax.experimental.pallas import tpu_sc as plsc

NC = 2
NS = 16
NW = NC * NS
CB = 128

def _make_deg(np_rows, e):
    et = e // NW
    kf = et // CB
    tail = et - kf * CB
    rpt = np_rows // NS
    mesh = plsc.VectorSubcoreMesh(core_axis_name="c", subcore_axis_name="s")

    @functools.partial(
        pl.kernel,
        out_type=jax.ShapeDtypeStruct((NC, np_rows, 16), jnp.float32),
        mesh=mesh,
        compiler_params=pltpu.CompilerParams(use_tc_tiling_on_sc=False),
        scratch_types=[
            pltpu.VMEM((et,), jnp.int32),
            pltpu.VMEM((CB, 16), jnp.float32),
            pltpu.VMEM_SHARED((np_rows, 16), jnp.float32),
            pltpu.SemaphoreType.DMA,
        ],
    )
    def deg_kernel(ei_hbm, zeros_hbm, ones_hbm, out_hbm, dst_v, ones_v, acc,
                   sem):
        c = lax.axis_index("c")
        s = lax.axis_index("s")
        wid = c * NS + s
        pltpu.sync_copy(zeros_hbm.at[pl.ds(s * rpt, rpt)],
                        acc.at[pl.ds(s * rpt, rpt)])
        pltpu.sync_copy(ei_hbm.at[1, pl.ds(wid * et, et)], dst_v)
        pltpu.sync_copy(ones_hbm, ones_v)
        plsc.subcore_barrier()

        nbd = 8

        def body(j, carry):
            pltpu.async_copy(ones_v, acc.at[dst_v.at[pl.ds(j * CB, CB)]],
                             sem, add=True)

            @pl.when(j >= nbd)
            def _():
                pltpu.make_async_copy(ones_v,
                                      acc.at[dst_v.at[pl.ds(0, CB)]],
                                      sem).wait()

            return carry

        lax.fori_loop(0, kf, body, 0)
        for _ in range(min(nbd, kf)):
            pltpu.make_async_copy(ones_v, acc.at[dst_v.at[pl.ds(0, CB)]],
                                  sem).wait()
        if tail:
            pltpu.sync_copy(ones_v.at[pl.ds(0, tail)],
                            acc.at[dst_v.at[pl.ds(kf * CB, tail)]], add=True)
        plsc.subcore_barrier()
        pltpu.sync_copy(acc.at[pl.ds(s * rpt, rpt)],
                        out_hbm.at[c, pl.ds(s * rpt, rpt)])

    return deg_kernel


def _prescale_body(degp_ref, x_ref, g1_ref, dinv_ref):
    deg = degp_ref[0, :, 0:1] + degp_ref[1, :, 0:1] + 1.0
    dinv = lax.rsqrt(deg)
    d = x_ref.shape[1]
    g1_ref[0] = x_ref[:, : d // 2] * dinv
    g1_ref[1] = x_ref[:, d // 2:] * dinv
    dinv_ref[...] = jnp.broadcast_to(dinv, dinv_ref.shape)


def _ring_propagate(gh, src_v, dst_v, rows_v, acc, gsem, ssem, kf, tail, nb):
    for b in range(nb):
        pltpu.async_copy(gh.at[src_v.at[pl.ds(b * CB, CB)]], rows_v.at[b],
                         gsem)

    def body(j, carry):
        bj = lax.rem(j, nb)
        pltpu.make_async_copy(gh.at[src_v.at[pl.ds(bj * CB, CB)]],
                              rows_v.at[bj], gsem).wait()
        pltpu.async_copy(rows_v.at[bj], acc.at[dst_v.at[pl.ds(j * CB, CB)]],
                         ssem, add=True)
        nxt = j + (nb - 1)

        @pl.when((j >= 1) & (nxt < kf))
        def _():
            bp = lax.rem(nxt, nb)
            pltpu.make_async_copy(rows_v.at[bp],
                                  acc.at[dst_v.at[pl.ds(0, CB)]], ssem).wait()
            pltpu.async_copy(gh.at[src_v.at[pl.ds(nxt * CB, CB)]],
                             rows_v.at[bp], gsem)

        return carry

    lax.fori_loop(0, kf, body, 0)
    for _ in range(nb):
        pltpu.make_async_copy(rows_v.at[0], acc.at[dst_v.at[pl.ds(0, CB)]],
                              ssem).wait()
    if tail:
        t0 = kf * CB
        pltpu.async_copy(gh.at[src_v.at[pl.ds(t0, tail)]],
                         rows_v.at[0, pl.ds(0, tail)], gsem).wait()
        pltpu.sync_copy(rows_v.at[0, pl.ds(0, tail)],
                        acc.at[dst_v.at[pl.ds(t0, tail)]], add=True)


def _init_acc_from_g(acc, g_hbm, zeros_hbm, c, s, n, rpt, last, last_cnt):
    r0 = s * rpt

    @pl.when(s < last)
    def _():
        pltpu.sync_copy(g_hbm.at[c, pl.ds(r0, rpt)], acc.at[pl.ds(r0, rpt)])

    @pl.when(s == last)
    def _():
        pltpu.sync_copy(g_hbm.at[c, pl.ds(r0, last_cnt)],
                        acc.at[pl.ds(r0, last_cnt)])
        pltpu.sync_copy(zeros_hbm.at[pl.ds(0, rpt - last_cnt)],
                        acc.at[pl.ds(r0 + last_cnt, rpt - last_cnt)])


def _make_prop(n, np_rows, dh, e, nb):
    et = e // NS
    kf = et // CB
    tail = et - kf * CB
    rpt = np_rows // NS
    last = NS - 1
    last_cnt = n - last * rpt
    mesh = plsc.VectorSubcoreMesh(core_axis_name="c", subcore_axis_name="s")

    @functools.partial(
        pl.kernel,
        out_type=jax.ShapeDtypeStruct((NC, np_rows, dh), jnp.float32),
        mesh=mesh,
        compiler_params=pltpu.CompilerParams(use_tc_tiling_on_sc=False),
        scratch_types=[
            pltpu.VMEM((et,), jnp.int32),
            pltpu.VMEM((et,), jnp.int32),
            pltpu.VMEM((nb, CB, dh), jnp.float32),
            pltpu.VMEM_SHARED((np_rows, dh), jnp.float32),
            pltpu.SemaphoreType.DMA,
            pltpu.SemaphoreType.DMA,
        ],
    )
    def prop_kernel(ei_hbm, g_hbm, zeros_hbm, out_hbm,
                    src_v, dst_v, rows_v, acc, gsem, ssem):
        c = lax.axis_index("c")
        s = lax.axis_index("s")
        _init_acc_from_g(acc, g_hbm, zeros_hbm, c, s, n, rpt, last, last_cnt)
        pltpu.sync_copy(ei_hbm.at[0, pl.ds(s * et, et)], src_v)
        pltpu.sync_copy(ei_hbm.at[1, pl.ds(s * et, et)], dst_v)
        plsc.subcore_barrier()
        _ring_propagate(g_hbm.at[c], src_v, dst_v, rows_v, acc,
                        gsem, ssem, kf, tail, nb)
        plsc.subcore_barrier()
        pltpu.sync_copy(acc.at[pl.ds(s * rpt, rpt)],
                        out_hbm.at[c, pl.ds(s * rpt, rpt)])

    return prop_kernel


def _make_prop_final(n, np_rows, dh, e, nb):
    et = e // NS
    kf = et // CB
    tail = et - kf * CB
    rpt = np_rows // NS
    last = NS - 1
    last_cnt = n - last * rpt
    mesh = plsc.VectorSubcoreMesh(core_axis_name="c", subcore_axis_name="s")

    @functools.partial(
        pl.kernel,
        out_type=jax.ShapeDtypeStruct((NC, n, dh), jnp.float32),
        mesh=mesh,
        compiler_params=pltpu.CompilerParams(use_tc_tiling_on_sc=False),
        scratch_types=[
            pltpu.VMEM((et,), jnp.int32),
            pltpu.VMEM((et,), jnp.int32),
            pltpu.VMEM((nb, CB, dh), jnp.float32),
            pltpu.VMEM((rpt, dh), jnp.float32),
            pltpu.VMEM((rpt, 16), jnp.float32),
            pltpu.VMEM((dh,), jnp.float32),
            pltpu.VMEM_SHARED((np_rows, dh), jnp.float32),
            pltpu.SemaphoreType.DMA,
            pltpu.SemaphoreType.DMA,
        ],
    )
    def prop_kernel(ei_hbm, g_hbm, zeros_hbm, dinv_hbm, bias_hbm,
                    out_hbm, src_v, dst_v, rows_v, res_v, dinv_v,
                    bias_v, acc, gsem, ssem):
        c = lax.axis_index("c")
        s = lax.axis_index("s")
        _init_acc_from_g(acc, g_hbm, zeros_hbm, c, s, n, rpt, last, last_cnt)
        pltpu.sync_copy(ei_hbm.at[0, pl.ds(s * et, et)], src_v)
        pltpu.sync_copy(ei_hbm.at[1, pl.ds(s * et, et)], dst_v)
        pltpu.sync_copy(bias_hbm.at[c], bias_v)
        plsc.subcore_barrier()
        _ring_propagate(g_hbm.at[c], src_v, dst_v, rows_v, acc,
                        gsem, ssem, kf, tail, nb)
        plsc.subcore_barrier()

        r0 = s * rpt
        pltpu.sync_copy(acc.at[pl.ds(r0, rpt)], res_v)
        bias = bias_v[:]

        def fin(i, carry):
            for kk in range(dh // 16):
                sl = pl.ds(kk * 16, 16)
                res_v[i, sl] = res_v[i, sl] * dinv_v[i, pl.ds(0, 16)] + bias
            return carry

        @pl.when(s < last)
        def _():
            pltpu.sync_copy(dinv_hbm.at[pl.ds(r0, rpt)], dinv_v)
            lax.fori_loop(0, rpt, fin, 0)
            pltpu.sync_copy(res_v, out_hbm.at[c, pl.ds(r0, rpt)])

        @pl.when(s == last)
        def _():
            pltpu.sync_copy(dinv_hbm.at[pl.ds(r0, last_cnt)],
                            dinv_v.at[pl.ds(0, last_cnt)])
            lax.fori_loop(0, last_cnt, fin, 0)
            pltpu.sync_copy(res_v.at[pl.ds(0, last_cnt)],
                            out_hbm.at[c, pl.ds(r0, last_cnt)])

    return prop_kernel


def _mlp_body(dinv16_ref, s1_ref, w1_ref, b1_ref, w2_ref, g2_ref):
    dinv = dinv16_ref[:, 0:1]
    a1 = jnp.concatenate([s1_ref[0], s1_ref[1]], axis=1) * dinv
    h = jnp.dot(a1, w1_ref[...], preferred_element_type=jnp.float32)
    h = jnp.maximum(h + b1_ref[...], 0.0)
    t = jnp.dot(h, w2_ref[...], preferred_element_type=jnp.float32)
    g2 = t * dinv
    ch = t.shape[1] // 2
    g2_ref[0] = g2[:, :ch]
    g2_ref[1] = g2[:, ch:]


def kernel(x, edge_index, W1, b1, W2, b2):
    n, d = x.shape
    h_dim = W1.shape[1]
    c_dim = W2.shape[1]
    e = edge_index.shape[1]
    dh = d // 2

    np_rows = -(-n // (NS * 8)) * NS * 8

    z16 = jnp.zeros((np_rows, 16), jnp.float32)
    zd = jnp.zeros((np_rows // NS, dh), jnp.float32)
    zc = jnp.zeros((np_rows // NS, c_dim // 2), jnp.float32)
    ones = jnp.ones((CB, 16), jnp.float32)

    degp = _make_deg(np_rows, e)(edge_index, z16, ones)

    bn = 1000
    grid = (n // bn,)
    g1, dinv16 = pl.pallas_call(
        _prescale_body,
        grid=grid,
        in_specs=[
            pl.BlockSpec((2, bn, 16), lambda i: (0, i, 0)),
            pl.BlockSpec((bn, d), lambda i: (i, 0)),
        ],
        out_specs=[
            pl.BlockSpec((2, bn, dh), lambda i: (0, i, 0)),
            pl.BlockSpec((bn, 16), lambda i: (i, 0)),
        ],
        out_shape=[
            jax.ShapeDtypeStruct((2, n, dh), jnp.float32),
            jax.ShapeDtypeStruct((n, 16), jnp.float32),
        ],
    )(degp, x)

    s1 = _make_prop(n, np_rows, dh, e, 6)(edge_index, g1, zd)
    g2 = pl.pallas_call(
        _mlp_body,
        grid=grid,
        in_specs=[
            pl.BlockSpec((bn, 16), lambda i: (i, 0)),
            pl.BlockSpec((2, bn, dh), lambda i: (0, i, 0)),
            pl.BlockSpec((d, h_dim), lambda i: (0, 0)),
            pl.BlockSpec((1, h_dim), lambda i: (0, 0)),
            pl.BlockSpec((h_dim, c_dim), lambda i: (0, 0)),
        ],
        out_specs=pl.BlockSpec((2, bn, c_dim // 2), lambda i: (0, i, 0)),
        out_shape=jax.ShapeDtypeStruct((2, n, c_dim // 2), jnp.float32),
    )(dinv16, s1, W1, b1.reshape(1, h_dim), W2)

    b2h = b2.reshape(NC, c_dim // 2)
    outh = _make_prop_final(n, np_rows, c_dim // 2, e, 12)(
        edge_index, g2, zc, dinv16, b2h)
    return jnp.concatenate([outh[0], outh[1]], axis=1)
